# baseline (speedup 1.0000x reference)
import functools

import jax
import jax.numpy as jnp
from jax import lax
from jax.experimental import pallas as pl
from jax.experimental.pallas import tpu as pltpu
from jax.experimental.pallas import tpu_sc as plsc

NN = 10000
NE = 320000
DIN = 128
DH = 32
DO = 16

NC = 2
NS = 16
LANES = 16
NW = NC * NS
NNP = 10240
RPT = NNP // NS

EROWS = 2560
EPAD = EROWS * 128 - NE
RPW = EROWS // NW

_MESH = dict(core_axis_name="c", subcore_axis_name="s")


def _worker_id():
    return lax.axis_index("s") * NC + lax.axis_index("c")


DEG_G = 8
DEG_NG = RPW // DEG_G


@functools.partial(
    pl.kernel,
    out_type=jax.ShapeDtypeStruct((NW, NNP), jnp.float32),
    mesh=plsc.VectorSubcoreMesh(**_MESH),
    scratch_types=[
        pltpu.VMEM((NNP,), jnp.float32),
        pltpu.VMEM((2, DEG_G, 128), jnp.int32),
        pltpu.SemaphoreType.DMA,
        pltpu.SemaphoreType.DMA,
    ],
    compiler_params=pltpu.CompilerParams(needs_layout_passes=False),
)
def _deg_kernel(dstm, out_hbm, hist, didx, lsem0, lsem1):
    wid = _worker_id()
    lsem = (lsem0, lsem1)
    zero16 = jnp.zeros((LANES,), jnp.float32)
    ones16 = jnp.ones((LANES,), jnp.float32)

    def zbody(i, c):
        hist[pl.ds(i * LANES, LANES)] = zero16
        return c

    lax.fori_loop(0, NNP // LANES, zbody, 0, unroll=8)

    lo = wid * RPW
    pltpu.sync_copy(dstm.at[pl.ds(lo, DEG_G)], didx.at[0])
    descs = {}
    for g in range(DEG_NG):
        b = g & 1
        if g > 0:
            descs[g].wait()
        if g + 1 < DEG_NG:
            descs[g + 1] = pltpu.async_copy(
                dstm.at[pl.ds(lo + (g + 1) * DEG_G, DEG_G)],
                didx.at[1 - b], lsem[1 - b])
        for j in range(DEG_G):
            for k in range(128 // LANES):
                idx = didx[b, j, pl.ds(k * LANES, LANES)]
                plsc.addupdate_scatter(hist, [idx], ones16)
    pltpu.sync_copy(hist, out_hbm.at[wid])


AGG_G = 8
NBUF = 3
AGG_NG0 = 15
AGG_NG1 = 5
NZCH = 4
ZR = RPT // NZCH


def _make_agg(d):
    @functools.partial(
        pl.kernel,
        out_type=jax.ShapeDtypeStruct((NC, NNP, d), jnp.float32),
        mesh=plsc.VectorSubcoreMesh(**_MESH),
        scratch_types=[
            pltpu.VMEM((NBUF, AGG_G, 128), jnp.int32),
            pltpu.VMEM((NBUF, AGG_G, 128), jnp.int32),
            pltpu.VMEM((NBUF, AGG_G, 128, d), jnp.float32),
            pltpu.VMEM_SHARED((NNP, d), jnp.float32),
            [pltpu.SemaphoreType.DMA] * NBUF,
            [pltpu.SemaphoreType.DMA] * NBUF,
            pltpu.SemaphoreType.DMA,
        ],
        compiler_params=pltpu.CompilerParams(use_tc_tiling_on_sc=False),
    )
    def agg(table, srcm, dstm, zeros_hbm, out_hbm, sidx, didx, rows,
            accum, gsem, ssem, zsem):
        cid = lax.axis_index("c")
        sid = lax.axis_index("s")

        zds = [
            pltpu.async_copy(
                zeros_hbm.at[pl.ds(sid * RPT + k * ZR, ZR)],
                accum.at[pl.ds(sid * RPT + k * ZR, ZR)], zsem)
            for k in range(NZCH)
        ]

        def edge_pipeline(lo, ng):
            def load_group(g, b):
                pltpu.sync_copy(srcm.at[pl.ds(lo + g * AGG_G, AGG_G)],
                                sidx.at[b])
                pltpu.sync_copy(dstm.at[pl.ds(lo + g * AGG_G, AGG_G)],
                                didx.at[b])

            def start_gathers(g, b):
                return [
                    pltpu.async_copy(table.at[sidx.at[b, j]], rows.at[b, j],
                                     gsem[b])
                    for j in range(AGG_G)
                ]

            load_group(0, 0)
            gd = {0: start_gathers(0, 0)}
            sd = [None] * NBUF
            for g in range(ng):
                b = g % NBUF
                if g == 0:
                    for zd in zds:
                        zd.wait()
                    plsc.subcore_barrier()
                if g + 1 < ng:
                    nb = (g + 1) % NBUF
                    if sd[nb] is not None:
                        for dsc in sd[nb]:
                            dsc.wait()
                        sd[nb] = None
                    load_group(g + 1, nb)
                    gd[g + 1] = start_gathers(g + 1, nb)
                scs = []
                for j in range(AGG_G):
                    gd[g][j].wait()
                    scs.append(pltpu.async_copy(
                        rows.at[b, j], accum.at[didx.at[b, j]], ssem[b],
                        add=True))
                sd[b] = scs
            for b in range(NBUF):
                if sd[b] is not None:
                    for dsc in sd[b]:
                        dsc.wait()

        @pl.when(cid == 0)
        def _():
            edge_pipeline(sid * (AGG_NG0 * AGG_G), AGG_NG0)

        @pl.when(cid == 1)
        def _():
            edge_pipeline(16 * AGG_NG0 * AGG_G + sid * (AGG_NG1 * AGG_G),
                          AGG_NG1)

        plsc.subcore_barrier()
        wds = [
            pltpu.async_copy(
                accum.at[pl.ds(sid * RPT + k * ZR, ZR)],
                out_hbm.at[cid, pl.ds(sid * RPT + k * ZR, ZR)], zsem)
            for k in range(NZCH)
        ]
        for wd in wds:
            wd.wait()

    return agg


_agg32 = _make_agg(DH)
_agg16 = _make_agg(DO)


def _tc_prep(parts, x, W1):
    def body(parts_ref, x_ref, w_ref, hp_ref, dinv_ref):
        ones = jnp.ones((NW, 1), jnp.float32)
        deg = lax.dot_general(
            parts_ref[...], ones, (((0,), (0,)), ((), ())),
            preferred_element_type=jnp.float32)
        dinv = lax.rsqrt(deg[:NN] + 1.0)
        h = jnp.dot(x_ref[...], w_ref[...], preferred_element_type=jnp.float32)
        hp_ref[:NN] = h * dinv
        hp_ref[NN:] = jnp.zeros((NNP - NN, DH), jnp.float32)
        dinv_ref[...] = dinv

    return pl.pallas_call(
        body,
        out_shape=[
            jax.ShapeDtypeStruct((NNP, DH), jnp.float32),
            jax.ShapeDtypeStruct((NN, 1), jnp.float32),
        ],
    )(parts, x, W1)


def _tc_mid(acc, hp1, dinv, b1, W2):
    def body(acc_ref, hp_ref, dinv_ref, b_ref, w_ref, out_ref):
        acc = acc_ref[...]
        s = acc[0, :NN] + acc[1, :NN] + hp_ref[:NN]
        h1 = jnp.maximum(dinv_ref[...] * s + b_ref[...], 0.0)
        out_ref[:NN] = (
            jnp.dot(h1, w_ref[...], preferred_element_type=jnp.float32)
            * dinv_ref[...])
        out_ref[NN:] = jnp.zeros((NNP - NN, DO), jnp.float32)

    return pl.pallas_call(
        body,
        out_shape=jax.ShapeDtypeStruct((NNP, DO), jnp.float32),
    )(acc, hp1, dinv, b1, W2)


def _tc_final(acc, hp2, dinv, b2):
    def body(acc_ref, hp_ref, dinv_ref, b_ref, out_ref):
        acc = acc_ref[...]
        z = dinv_ref[...] * (acc[0, :NN] + acc[1, :NN] + hp_ref[:NN]) + b_ref[...]
        m = jnp.max(z, axis=1, keepdims=True)
        lse = jnp.log(jnp.sum(jnp.exp(z - m), axis=1, keepdims=True)) + m
        out_ref[...] = z - lse

    return pl.pallas_call(
        body,
        out_shape=jax.ShapeDtypeStruct((NN, DO), jnp.float32),
    )(acc, hp2, dinv, b2)


def kernel(x, edge_index, W1, b1, W2, b2):
    pad = jnp.full((EPAD,), NN, jnp.int32)
    srcm = jnp.concatenate([edge_index[0], pad]).reshape(EROWS, 128)
    dstm = jnp.concatenate([edge_index[1], pad]).reshape(EROWS, 128)
    parts = _deg_kernel(dstm)
    hp1, dinv = _tc_prep(parts, x, W1)
    z32 = jnp.zeros((NNP, DH), jnp.float32)
    z16 = jnp.zeros((NNP, DO), jnp.float32)
    acc1 = _agg32(hp1, srcm, dstm, z32)
    hp2 = _tc_mid(acc1, hp1, dinv, b1.reshape(1, DH), W2)
    acc2 = _agg16(hp2, srcm, dstm, z16)
    return _tc_final(acc2, hp2, dinv, b2.reshape(1, DO))


# speedup vs baseline: 1.0382x; 1.0382x over previous
import functools

import jax
import jax.numpy as jnp
from jax import lax
from jax.experimental import pallas as pl
from jax.experimental.pallas import tpu as pltpu
from jax.experimental.pallas import tpu_sc as plsc

NN = 10000
NE = 320000
DIN = 128
DH = 32
DO = 16

NC = 2
NS = 16
LANES = 16
NW = NC * NS
NNP = 10240
RPT = NNP // NS

EROWS = 2560
EPAD = EROWS * 128 - NE
RPW = EROWS // NW

_MESH = dict(core_axis_name="c", subcore_axis_name="s")


def _worker_id():
    return lax.axis_index("s") * NC + lax.axis_index("c")


DEG_G = 8
DEG_NG = RPW // DEG_G


@functools.partial(
    pl.kernel,
    out_type=jax.ShapeDtypeStruct((NW, NNP), jnp.float32),
    mesh=plsc.VectorSubcoreMesh(**_MESH),
    scratch_types=[
        pltpu.VMEM((NNP,), jnp.float32),
        pltpu.VMEM((2, DEG_G, 128), jnp.int32),
        pltpu.SemaphoreType.DMA,
        pltpu.SemaphoreType.DMA,
    ],
    compiler_params=pltpu.CompilerParams(needs_layout_passes=False),
)
def _deg_kernel(dstm, out_hbm, hist, didx, lsem0, lsem1):
    wid = _worker_id()
    lsem = (lsem0, lsem1)
    zero16 = jnp.zeros((LANES,), jnp.float32)
    ones16 = jnp.ones((LANES,), jnp.float32)

    def zbody(i, c):
        hist[pl.ds(i * LANES, LANES)] = zero16
        return c

    lax.fori_loop(0, NNP // LANES, zbody, 0, unroll=8)

    lo = wid * RPW
    pltpu.sync_copy(dstm.at[pl.ds(lo, DEG_G)], didx.at[0])
    descs = {}
    for g in range(DEG_NG):
        b = g & 1
        if g > 0:
            descs[g].wait()
        if g + 1 < DEG_NG:
            descs[g + 1] = pltpu.async_copy(
                dstm.at[pl.ds(lo + (g + 1) * DEG_G, DEG_G)],
                didx.at[1 - b], lsem[1 - b])
        for j in range(DEG_G):
            for k in range(128 // LANES):
                idx = didx[b, j, pl.ds(k * LANES, LANES)]
                plsc.addupdate_scatter(hist, [idx], ones16)
    pltpu.sync_copy(hist, out_hbm.at[wid])


AGG_G = 8
NZCH = 4
ZR = RPT // NZCH


def _make_agg(d, ng0, ng1):
    @functools.partial(
        pl.kernel,
        out_type=jax.ShapeDtypeStruct((NC, NNP, d), jnp.float32),
        mesh=plsc.VectorSubcoreMesh(**_MESH),
        scratch_types=[
            pltpu.VMEM((2, AGG_G, 128), jnp.int32),
            pltpu.VMEM((2, AGG_G, 128), jnp.int32),
            pltpu.VMEM((2, AGG_G, 128, d), jnp.float32),
            pltpu.VMEM((RPT, d), jnp.float32),
            pltpu.VMEM_SHARED((NNP, d), jnp.float32),
            [pltpu.SemaphoreType.DMA] * 2,
            [pltpu.SemaphoreType.DMA] * 2,
            pltpu.SemaphoreType.DMA,
        ],
        compiler_params=pltpu.CompilerParams(use_tc_tiling_on_sc=False),
    )
    def agg(table, srcm, dstm, out_hbm, sidx, didx, rows, bounce, accum,
            gsem, ssem, zsem):
        cid = lax.axis_index("c")
        sid = lax.axis_index("s")
        zero16 = jnp.zeros((LANES,), jnp.float32)

        def zbody(i, carry):
            for j in range(d // LANES):
                bounce[i, pl.ds(j * LANES, LANES)] = zero16
            return carry

        lax.fori_loop(0, RPT, zbody, 0, unroll=8)
        zds = [
            pltpu.async_copy(
                bounce.at[pl.ds(k * ZR, ZR)],
                accum.at[pl.ds(sid * RPT + k * ZR, ZR)], zsem)
            for k in range(NZCH)
        ]

        def edge_pipeline(lo, ng):
            def load_group(g, b):
                pltpu.sync_copy(srcm.at[pl.ds(lo + g * AGG_G, AGG_G)],
                                sidx.at[b])
                pltpu.sync_copy(dstm.at[pl.ds(lo + g * AGG_G, AGG_G)],
                                didx.at[b])

            def start_gathers(g, b):
                return [
                    pltpu.async_copy(table.at[sidx.at[b, j]], rows.at[b, j],
                                     gsem[b])
                    for j in range(AGG_G)
                ]

            load_group(0, 0)
            gd = {0: start_gathers(0, 0)}
            sd = [None, None]
            for g in range(ng):
                b = g & 1
                nb = 1 - b
                if g == 0:
                    for zd in zds:
                        zd.wait()
                    plsc.subcore_barrier()
                if g + 1 < ng:
                    if sd[nb] is not None:
                        for dsc in sd[nb]:
                            dsc.wait()
                        sd[nb] = None
                    load_group(g + 1, nb)
                    gd[g + 1] = start_gathers(g + 1, nb)
                scs = []
                for j in range(AGG_G):
                    gd[g][j].wait()
                    scs.append(pltpu.async_copy(
                        rows.at[b, j], accum.at[didx.at[b, j]], ssem[b],
                        add=True))
                sd[b] = scs
            for b in (0, 1):
                if sd[b] is not None:
                    for dsc in sd[b]:
                        dsc.wait()

        @pl.when(cid == 0)
        def _():
            edge_pipeline(sid * (ng0 * AGG_G), ng0)

        @pl.when(cid == 1)
        def _():
            edge_pipeline(16 * ng0 * AGG_G + sid * (ng1 * AGG_G), ng1)

        plsc.subcore_barrier()
        wds = [
            pltpu.async_copy(
                accum.at[pl.ds(sid * RPT + k * ZR, ZR)],
                out_hbm.at[cid, pl.ds(sid * RPT + k * ZR, ZR)], zsem)
            for k in range(NZCH)
        ]
        for wd in wds:
            wd.wait()

    return agg


_agg32 = _make_agg(DH, 15, 5)
_agg16 = _make_agg(DO, 18, 2)


def _tc_prep(parts, x, W1):
    def body(parts_ref, x_ref, w_ref, hp_ref, dinv_ref):
        ones = jnp.ones((NW, 1), jnp.float32)
        deg = lax.dot_general(
            parts_ref[...], ones, (((0,), (0,)), ((), ())),
            preferred_element_type=jnp.float32)
        dinv = lax.rsqrt(deg[:NN] + 1.0)
        h = jnp.dot(x_ref[...], w_ref[...], preferred_element_type=jnp.float32)
        hp_ref[:NN] = h * dinv
        hp_ref[NN:] = jnp.zeros((NNP - NN, DH), jnp.float32)
        dinv_ref[...] = dinv

    return pl.pallas_call(
        body,
        out_shape=[
            jax.ShapeDtypeStruct((NNP, DH), jnp.float32),
            jax.ShapeDtypeStruct((NN, 1), jnp.float32),
        ],
    )(parts, x, W1)


def _tc_mid(acc, hp1, dinv, b1, W2):
    def body(acc_ref, hp_ref, dinv_ref, b_ref, w_ref, out_ref):
        acc = acc_ref[...]
        s = acc[0, :NN] + acc[1, :NN] + hp_ref[:NN]
        h1 = jnp.maximum(dinv_ref[...] * s + b_ref[...], 0.0)
        out_ref[:NN] = (
            jnp.dot(h1, w_ref[...], preferred_element_type=jnp.float32)
            * dinv_ref[...])
        out_ref[NN:] = jnp.zeros((NNP - NN, DO), jnp.float32)

    return pl.pallas_call(
        body,
        out_shape=jax.ShapeDtypeStruct((NNP, DO), jnp.float32),
    )(acc, hp1, dinv, b1, W2)


def _tc_final(acc, hp2, dinv, b2):
    def body(acc_ref, hp_ref, dinv_ref, b_ref, out_ref):
        acc = acc_ref[...]
        z = dinv_ref[...] * (acc[0, :NN] + acc[1, :NN] + hp_ref[:NN]) + b_ref[...]
        m = jnp.max(z, axis=1, keepdims=True)
        lse = jnp.log(jnp.sum(jnp.exp(z - m), axis=1, keepdims=True)) + m
        out_ref[...] = z - lse

    return pl.pallas_call(
        body,
        out_shape=jax.ShapeDtypeStruct((NN, DO), jnp.float32),
    )(acc, hp2, dinv, b2)


def kernel(x, edge_index, W1, b1, W2, b2):
    pad = jnp.full((EPAD,), NN, jnp.int32)
    srcm = jnp.concatenate([edge_index[0], pad]).reshape(EROWS, 128)
    dstm = jnp.concatenate([edge_index[1], pad]).reshape(EROWS, 128)
    parts = _deg_kernel(dstm)
    hp1, dinv = _tc_prep(parts, x, W1)
    acc1 = _agg32(hp1, srcm, dstm)
    hp2 = _tc_mid(acc1, hp1, dinv, b1.reshape(1, DH), W2)
    acc2 = _agg16(hp2, srcm, dstm)
    return _tc_final(acc2, hp2, dinv, b2.reshape(1, DO))


# speedup vs baseline: 1.0399x; 1.0017x over previous
import functools

import jax
import jax.numpy as jnp
from jax import lax
from jax.experimental import pallas as pl
from jax.experimental.pallas import tpu as pltpu
from jax.experimental.pallas import tpu_sc as plsc

NN = 10000
NE = 320000
DIN = 128
DH = 32
DO = 16

NC = 2
NS = 16
LANES = 16
NW = NC * NS
NNP = 10240
RPT = NNP // NS

EROWS = 2560
EPAD = EROWS * 128 - NE
RPW = EROWS // NW

_MESH = dict(core_axis_name="c", subcore_axis_name="s")


def _worker_id():
    return lax.axis_index("s") * NC + lax.axis_index("c")


DEG_G = 8
DEG_NG = RPW // DEG_G


@functools.partial(
    pl.kernel,
    out_type=jax.ShapeDtypeStruct((NW, NNP), jnp.float32),
    mesh=plsc.VectorSubcoreMesh(**_MESH),
    scratch_types=[
        pltpu.VMEM((NNP,), jnp.float32),
        pltpu.VMEM((2, DEG_G, 128), jnp.int32),
        pltpu.SemaphoreType.DMA,
        pltpu.SemaphoreType.DMA,
    ],
    compiler_params=pltpu.CompilerParams(needs_layout_passes=False),
)
def _deg_kernel(dstm, out_hbm, hist, didx, lsem0, lsem1):
    wid = _worker_id()
    lsem = (lsem0, lsem1)
    zero16 = jnp.zeros((LANES,), jnp.float32)
    ones16 = jnp.ones((LANES,), jnp.float32)

    def zbody(i, c):
        hist[pl.ds(i * LANES, LANES)] = zero16
        return c

    lax.fori_loop(0, NNP // LANES, zbody, 0, unroll=8)

    lo = wid * RPW
    pltpu.sync_copy(dstm.at[pl.ds(lo, DEG_G)], didx.at[0])
    descs = {}
    for g in range(DEG_NG):
        b = g & 1
        if g > 0:
            descs[g].wait()
        if g + 1 < DEG_NG:
            descs[g + 1] = pltpu.async_copy(
                dstm.at[pl.ds(lo + (g + 1) * DEG_G, DEG_G)],
                didx.at[1 - b], lsem[1 - b])
        for j in range(DEG_G):
            for k in range(128 // LANES):
                idx = didx[b, j, pl.ds(k * LANES, LANES)]
                plsc.addupdate_scatter(hist, [idx], ones16)
    pltpu.sync_copy(hist, out_hbm.at[wid])


AGG_G = 8
NZCH = 4
ZR = RPT // NZCH


def _make_agg(d, ng0, ng1):
    @functools.partial(
        pl.kernel,
        out_type=jax.ShapeDtypeStruct((NC, NNP, d), jnp.float32),
        mesh=plsc.VectorSubcoreMesh(**_MESH),
        scratch_types=[
            pltpu.VMEM((2, AGG_G, 128), jnp.int32),
            pltpu.VMEM((2, AGG_G, 128), jnp.int32),
            pltpu.VMEM((2, AGG_G, 128, d), jnp.float32),
            pltpu.VMEM((RPT, d), jnp.float32),
            pltpu.VMEM_SHARED((NNP, d), jnp.float32),
            [pltpu.SemaphoreType.DMA] * 2,
            [pltpu.SemaphoreType.DMA] * 2,
            pltpu.SemaphoreType.DMA,
        ],
        compiler_params=pltpu.CompilerParams(use_tc_tiling_on_sc=False),
    )
    def agg(table, srcm, dstm, out_hbm, sidx, didx, rows, bounce, accum,
            gsem, ssem, zsem):
        cid = lax.axis_index("c")
        sid = lax.axis_index("s")
        zero16 = jnp.zeros((LANES,), jnp.float32)

        def zbody(i, carry):
            for j in range(d // LANES):
                bounce[i, pl.ds(j * LANES, LANES)] = zero16
            return carry

        lax.fori_loop(0, RPT, zbody, 0, unroll=8)
        zds = [
            pltpu.async_copy(
                bounce.at[pl.ds(k * ZR, ZR)],
                accum.at[pl.ds(sid * RPT + k * ZR, ZR)], zsem)
            for k in range(NZCH)
        ]

        def edge_pipeline(lo, ng):
            def load_group(g, b):
                pltpu.sync_copy(srcm.at[pl.ds(lo + g * AGG_G, AGG_G)],
                                sidx.at[b])
                pltpu.sync_copy(dstm.at[pl.ds(lo + g * AGG_G, AGG_G)],
                                didx.at[b])

            def start_gathers(g, b):
                return [
                    pltpu.async_copy(table.at[sidx.at[b, j]], rows.at[b, j],
                                     gsem[b])
                    for j in range(AGG_G)
                ]

            load_group(0, 0)
            gd = {0: start_gathers(0, 0)}
            sd = [None, None]
            for g in range(ng):
                b = g & 1
                nb = 1 - b
                if g == 0:
                    for zd in zds:
                        zd.wait()
                    plsc.subcore_barrier()
                if g + 1 < ng:
                    if sd[nb] is not None:
                        for dsc in sd[nb]:
                            dsc.wait()
                        sd[nb] = None
                    load_group(g + 1, nb)
                    gd[g + 1] = start_gathers(g + 1, nb)
                scs = []
                for j in range(AGG_G):
                    gd[g][j].wait()
                    scs.append(pltpu.async_copy(
                        rows.at[b, j], accum.at[didx.at[b, j]], ssem[b],
                        add=True))
                sd[b] = scs
            for b in (0, 1):
                if sd[b] is not None:
                    for dsc in sd[b]:
                        dsc.wait()

        @pl.when(cid == 0)
        def _():
            edge_pipeline(sid * (ng0 * AGG_G), ng0)

        @pl.when(cid == 1)
        def _():
            edge_pipeline(16 * ng0 * AGG_G + sid * (ng1 * AGG_G), ng1)

        plsc.subcore_barrier()
        wds = [
            pltpu.async_copy(
                accum.at[pl.ds(sid * RPT + k * ZR, ZR)],
                out_hbm.at[cid, pl.ds(sid * RPT + k * ZR, ZR)], zsem)
            for k in range(NZCH)
        ]
        for wd in wds:
            wd.wait()

    return agg


_agg32 = _make_agg(DH, 17, 3)
_agg16 = _make_agg(DO, 18, 2)


def _tc_prep(parts, x, W1):
    def body(parts_ref, x_ref, w_ref, hp_ref, dinv_ref):
        ones = jnp.ones((NW, 1), jnp.float32)
        deg = lax.dot_general(
            parts_ref[...], ones, (((0,), (0,)), ((), ())),
            preferred_element_type=jnp.float32)
        dinv = lax.rsqrt(deg[:NN] + 1.0)
        h = jnp.dot(x_ref[...], w_ref[...], preferred_element_type=jnp.float32)
        hp_ref[:NN] = h * dinv
        hp_ref[NN:] = jnp.zeros((NNP - NN, DH), jnp.float32)
        dinv_ref[...] = dinv

    return pl.pallas_call(
        body,
        out_shape=[
            jax.ShapeDtypeStruct((NNP, DH), jnp.float32),
            jax.ShapeDtypeStruct((NN, 1), jnp.float32),
        ],
    )(parts, x, W1)


def _tc_mid(acc, hp1, dinv, b1, W2):
    def body(acc_ref, hp_ref, dinv_ref, b_ref, w_ref, out_ref):
        acc = acc_ref[...]
        s = acc[0, :NN] + acc[1, :NN] + hp_ref[:NN]
        h1 = jnp.maximum(dinv_ref[...] * s + b_ref[...], 0.0)
        out_ref[:NN] = (
            jnp.dot(h1, w_ref[...], preferred_element_type=jnp.float32)
            * dinv_ref[...])
        out_ref[NN:] = jnp.zeros((NNP - NN, DO), jnp.float32)

    return pl.pallas_call(
        body,
        out_shape=jax.ShapeDtypeStruct((NNP, DO), jnp.float32),
    )(acc, hp1, dinv, b1, W2)


def _tc_final(acc, hp2, dinv, b2):
    def body(acc_ref, hp_ref, dinv_ref, b_ref, out_ref):
        acc = acc_ref[...]
        z = dinv_ref[...] * (acc[0, :NN] + acc[1, :NN] + hp_ref[:NN]) + b_ref[...]
        m = jnp.max(z, axis=1, keepdims=True)
        lse = jnp.log(jnp.sum(jnp.exp(z - m), axis=1, keepdims=True)) + m
        out_ref[...] = z - lse

    return pl.pallas_call(
        body,
        out_shape=jax.ShapeDtypeStruct((NN, DO), jnp.float32),
    )(acc, hp2, dinv, b2)


def kernel(x, edge_index, W1, b1, W2, b2):
    pad = jnp.full((EPAD,), NN, jnp.int32)
    srcm = jnp.concatenate([edge_index[0], pad]).reshape(EROWS, 128)
    dstm = jnp.concatenate([edge_index[1], pad]).reshape(EROWS, 128)
    parts = _deg_kernel(dstm)
    hp1, dinv = _tc_prep(parts, x, W1)
    acc1 = _agg32(hp1, srcm, dstm)
    hp2 = _tc_mid(acc1, hp1, dinv, b1.reshape(1, DH), W2)
    acc2 = _agg16(hp2, srcm, dstm)
    return _tc_final(acc2, hp2, dinv, b2.reshape(1, DO))


# speedup vs baseline: 1.0427x; 1.0027x over previous
import functools

import jax
import jax.numpy as jnp
from jax import lax
from jax.experimental import pallas as pl
from jax.experimental.pallas import tpu as pltpu
from jax.experimental.pallas import tpu_sc as plsc

NN = 10000
NE = 320000
DIN = 128
DH = 32
DO = 16

NC = 2
NS = 16
LANES = 16
NW = NC * NS
NNP = 10240
RPT = NNP // NS

EROWS = 2560
EPAD = EROWS * 128 - NE
RPW = EROWS // NW

_MESH = dict(core_axis_name="c", subcore_axis_name="s")


def _worker_id():
    return lax.axis_index("s") * NC + lax.axis_index("c")


DEG_G = 8
DEG_NG0 = 12
DEG_NG1 = 8


@functools.partial(
    pl.kernel,
    out_type=jax.ShapeDtypeStruct((NW, NNP), jnp.float32),
    mesh=plsc.VectorSubcoreMesh(**_MESH),
    scratch_types=[
        pltpu.VMEM((NNP,), jnp.float32),
        pltpu.VMEM((2, DEG_G, 128), jnp.int32),
        pltpu.SemaphoreType.DMA,
        pltpu.SemaphoreType.DMA,
    ],
    compiler_params=pltpu.CompilerParams(needs_layout_passes=False),
)
def _deg_kernel(dstm, out_hbm, hist, didx, lsem0, lsem1):
    wid = _worker_id()
    cid = lax.axis_index("c")
    sid = lax.axis_index("s")
    lsem = (lsem0, lsem1)
    zero16 = jnp.zeros((LANES,), jnp.float32)
    ones16 = jnp.ones((LANES,), jnp.float32)

    def zbody(i, c):
        hist[pl.ds(i * LANES, LANES)] = zero16
        return c

    lax.fori_loop(0, NNP // LANES, zbody, 0, unroll=8)

    def hist_pipeline(lo, ng):
        pltpu.sync_copy(dstm.at[pl.ds(lo, DEG_G)], didx.at[0])
        descs = {}
        for g in range(ng):
            b = g & 1
            if g > 0:
                descs[g].wait()
            if g + 1 < ng:
                descs[g + 1] = pltpu.async_copy(
                    dstm.at[pl.ds(lo + (g + 1) * DEG_G, DEG_G)],
                    didx.at[1 - b], lsem[1 - b])
            for j in range(DEG_G):
                for k in range(128 // LANES):
                    idx = didx[b, j, pl.ds(k * LANES, LANES)]
                    plsc.addupdate_scatter(hist, [idx], ones16)

    @pl.when(cid == 0)
    def _():
        hist_pipeline(sid * (DEG_NG0 * DEG_G), DEG_NG0)

    @pl.when(cid == 1)
    def _():
        hist_pipeline(16 * DEG_NG0 * DEG_G + sid * (DEG_NG1 * DEG_G), DEG_NG1)

    pltpu.sync_copy(hist, out_hbm.at[wid])


NZCH = 4
ZR = RPT // NZCH


def _make_agg(d, ng0, ng1, ag):
    @functools.partial(
        pl.kernel,
        out_type=jax.ShapeDtypeStruct((NC, NNP, d), jnp.float32),
        mesh=plsc.VectorSubcoreMesh(**_MESH),
        scratch_types=[
            pltpu.VMEM((2, ag, 128), jnp.int32),
            pltpu.VMEM((2, ag, 128), jnp.int32),
            pltpu.VMEM((2, ag, 128, d), jnp.float32),
            pltpu.VMEM((RPT, d), jnp.float32),
            pltpu.VMEM_SHARED((NNP, d), jnp.float32),
            [pltpu.SemaphoreType.DMA] * 2,
            [pltpu.SemaphoreType.DMA] * 2,
            pltpu.SemaphoreType.DMA,
        ],
        compiler_params=pltpu.CompilerParams(use_tc_tiling_on_sc=False),
    )
    def agg(table, srcm, dstm, out_hbm, sidx, didx, rows, bounce, accum,
            gsem, ssem, zsem):
        cid = lax.axis_index("c")
        sid = lax.axis_index("s")
        zero16 = jnp.zeros((LANES,), jnp.float32)

        def zbody(i, carry):
            for j in range(d // LANES):
                bounce[i, pl.ds(j * LANES, LANES)] = zero16
            return carry

        lax.fori_loop(0, RPT, zbody, 0, unroll=8)
        zds = [
            pltpu.async_copy(
                bounce.at[pl.ds(k * ZR, ZR)],
                accum.at[pl.ds(sid * RPT + k * ZR, ZR)], zsem)
            for k in range(NZCH)
        ]

        def edge_pipeline(lo, ng):
            def load_group(g, b):
                pltpu.sync_copy(srcm.at[pl.ds(lo + g * ag, ag)],
                                sidx.at[b])
                pltpu.sync_copy(dstm.at[pl.ds(lo + g * ag, ag)],
                                didx.at[b])

            def start_gathers(g, b):
                return [
                    pltpu.async_copy(table.at[sidx.at[b, j]], rows.at[b, j],
                                     gsem[b])
                    for j in range(ag)
                ]

            load_group(0, 0)
            gd = {0: start_gathers(0, 0)}
            sd = [None, None]
            for g in range(ng):
                b = g & 1
                nb = 1 - b
                if g == 0:
                    for zd in zds:
                        zd.wait()
                    plsc.subcore_barrier()
                if g + 1 < ng:
                    if sd[nb] is not None:
                        for dsc in sd[nb]:
                            dsc.wait()
                        sd[nb] = None
                    load_group(g + 1, nb)
                    gd[g + 1] = start_gathers(g + 1, nb)
                scs = []
                for j in range(ag):
                    gd[g][j].wait()
                    scs.append(pltpu.async_copy(
                        rows.at[b, j], accum.at[didx.at[b, j]], ssem[b],
                        add=True))
                sd[b] = scs
            for b in (0, 1):
                if sd[b] is not None:
                    for dsc in sd[b]:
                        dsc.wait()

        @pl.when(cid == 0)
        def _():
            edge_pipeline(sid * (ng0 * ag), ng0)

        @pl.when(cid == 1)
        def _():
            edge_pipeline(16 * ng0 * ag + sid * (ng1 * ag), ng1)

        plsc.subcore_barrier()
        wds = [
            pltpu.async_copy(
                accum.at[pl.ds(sid * RPT + k * ZR, ZR)],
                out_hbm.at[cid, pl.ds(sid * RPT + k * ZR, ZR)], zsem)
            for k in range(NZCH)
        ]
        for wd in wds:
            wd.wait()

    return agg


_agg32 = _make_agg(DH, 17, 3, 8)
_agg16 = _make_agg(DO, 9, 1, 16)


def _tc_prep(parts, x, W1):
    def body(parts_ref, x_ref, w_ref, hp_ref, dinv_ref):
        ones = jnp.ones((NW, 1), jnp.float32)
        deg = lax.dot_general(
            parts_ref[...], ones, (((0,), (0,)), ((), ())),
            preferred_element_type=jnp.float32)
        dinv = lax.rsqrt(deg[:NN] + 1.0)
        h = jnp.dot(x_ref[...], w_ref[...], preferred_element_type=jnp.float32)
        hp_ref[:NN] = h * dinv
        hp_ref[NN:] = jnp.zeros((NNP - NN, DH), jnp.float32)
        dinv_ref[...] = dinv

    return pl.pallas_call(
        body,
        out_shape=[
            jax.ShapeDtypeStruct((NNP, DH), jnp.float32),
            jax.ShapeDtypeStruct((NN, 1), jnp.float32),
        ],
    )(parts, x, W1)


def _tc_mid(acc, hp1, dinv, b1, W2):
    def body(acc_ref, hp_ref, dinv_ref, b_ref, w_ref, out_ref):
        acc = acc_ref[...]
        s = acc[0, :NN] + acc[1, :NN] + hp_ref[:NN]
        h1 = jnp.maximum(dinv_ref[...] * s + b_ref[...], 0.0)
        out_ref[:NN] = (
            jnp.dot(h1, w_ref[...], preferred_element_type=jnp.float32)
            * dinv_ref[...])
        out_ref[NN:] = jnp.zeros((NNP - NN, DO), jnp.float32)

    return pl.pallas_call(
        body,
        out_shape=jax.ShapeDtypeStruct((NNP, DO), jnp.float32),
    )(acc, hp1, dinv, b1, W2)


def _tc_final(acc, hp2, dinv, b2):
    def body(acc_ref, hp_ref, dinv_ref, b_ref, out_ref):
        acc = acc_ref[...]
        z = (dinv_ref[...] * (acc[0, :NN] + acc[1, :NN] + hp_ref[:NN])
             + b_ref[...])
        m = jnp.max(z, axis=1, keepdims=True)
        lse = jnp.log(jnp.sum(jnp.exp(z - m), axis=1, keepdims=True)) + m
        out_ref[...] = z - lse

    return pl.pallas_call(
        body,
        out_shape=jax.ShapeDtypeStruct((NN, DO), jnp.float32),
    )(acc, hp2, dinv, b2)


def kernel(x, edge_index, W1, b1, W2, b2):
    pad = jnp.full((EPAD,), NN, jnp.int32)
    srcm = jnp.concatenate([edge_index[0], pad]).reshape(EROWS, 128)
    dstm = jnp.concatenate([edge_index[1], pad]).reshape(EROWS, 128)
    parts = _deg_kernel(dstm)
    hp1, dinv = _tc_prep(parts, x, W1)
    acc1 = _agg32(hp1, srcm, dstm)
    hp2 = _tc_mid(acc1, hp1, dinv, b1.reshape(1, DH), W2)
    acc2 = _agg16(hp2, srcm, dstm)
    return _tc_final(acc2, hp2, dinv, b2.reshape(1, DO))


# speedup vs baseline: 1.0590x; 1.0157x over previous
import functools

import jax
import jax.numpy as jnp
from jax import lax
from jax.experimental import pallas as pl
from jax.experimental.pallas import tpu as pltpu
from jax.experimental.pallas import tpu_sc as plsc

NN = 10000
NE = 320000
DIN = 128
DH = 32
DO = 16

NC = 2
NS = 16
LANES = 16
NW = NC * NS
NNP = 10240
RPT = NNP // NS

EROWS = 2560
EPAD = EROWS * 128 - NE
RPW = EROWS // NW

_MESH = dict(core_axis_name="c", subcore_axis_name="s")


def _worker_id():
    return lax.axis_index("s") * NC + lax.axis_index("c")


DEG_G = 8
DEG_NG0 = 12
DEG_NG1 = 8


@functools.partial(
    pl.kernel,
    out_type=jax.ShapeDtypeStruct((NW, NNP), jnp.float32),
    mesh=plsc.VectorSubcoreMesh(**_MESH),
    scratch_types=[
        pltpu.VMEM((NNP,), jnp.float32),
        pltpu.VMEM((2, DEG_G, 128), jnp.int32),
        pltpu.SemaphoreType.DMA,
        pltpu.SemaphoreType.DMA,
    ],
    compiler_params=pltpu.CompilerParams(needs_layout_passes=False),
)
def _deg_kernel(dstm, out_hbm, hist, didx, lsem0, lsem1):
    wid = _worker_id()
    cid = lax.axis_index("c")
    sid = lax.axis_index("s")
    lsem = (lsem0, lsem1)
    zero16 = jnp.zeros((LANES,), jnp.float32)
    ones16 = jnp.ones((LANES,), jnp.float32)

    def zbody(i, c):
        hist[pl.ds(i * LANES, LANES)] = zero16
        return c

    lax.fori_loop(0, NNP // LANES, zbody, 0, unroll=8)

    def hist_pipeline(lo, ng):
        pltpu.sync_copy(dstm.at[pl.ds(lo, DEG_G)], didx.at[0])
        descs = {}
        for g in range(ng):
            b = g & 1
            if g > 0:
                descs[g].wait()
            if g + 1 < ng:
                descs[g + 1] = pltpu.async_copy(
                    dstm.at[pl.ds(lo + (g + 1) * DEG_G, DEG_G)],
                    didx.at[1 - b], lsem[1 - b])
            for j in range(DEG_G):
                for k in range(128 // LANES):
                    idx = didx[b, j, pl.ds(k * LANES, LANES)]
                    plsc.addupdate_scatter(hist, [idx], ones16)

    @pl.when(cid == 0)
    def _():
        hist_pipeline(sid * (DEG_NG0 * DEG_G), DEG_NG0)

    @pl.when(cid == 1)
    def _():
        hist_pipeline(16 * DEG_NG0 * DEG_G + sid * (DEG_NG1 * DEG_G), DEG_NG1)

    pltpu.sync_copy(hist, out_hbm.at[wid])


NZCH = 4
ZR = RPT // NZCH


def _make_agg(d, ng0, ng1, ag):
    @functools.partial(
        pl.kernel,
        out_type=jax.ShapeDtypeStruct((NC, NNP, d), jnp.float32),
        mesh=plsc.VectorSubcoreMesh(**_MESH),
        scratch_types=[
            pltpu.VMEM((2, ag, 128), jnp.int32),
            pltpu.VMEM((2, ag, 128), jnp.int32),
            pltpu.VMEM((2, ag, 128, d), jnp.float32),
            pltpu.VMEM((RPT, d), jnp.float32),
            pltpu.VMEM_SHARED((NNP, d), jnp.float32),
            [pltpu.SemaphoreType.DMA] * 2,
            [pltpu.SemaphoreType.DMA] * 2,
            pltpu.SemaphoreType.DMA,
        ],
        compiler_params=pltpu.CompilerParams(use_tc_tiling_on_sc=False),
    )
    def agg(table, srcm, dstm, out_hbm, sidx, didx, rows, bounce, accum,
            gsem, ssem, zsem):
        cid = lax.axis_index("c")
        sid = lax.axis_index("s")
        zero16 = jnp.zeros((LANES,), jnp.float32)

        def zbody(i, carry):
            for j in range(d // LANES):
                bounce[i, pl.ds(j * LANES, LANES)] = zero16
            return carry

        lax.fori_loop(0, RPT, zbody, 0, unroll=8)
        zds = [
            pltpu.async_copy(
                bounce.at[pl.ds(k * ZR, ZR)],
                accum.at[pl.ds(sid * RPT + k * ZR, ZR)], zsem)
            for k in range(NZCH)
        ]

        def edge_pipeline(lo, ng):
            def load_group(g, b):
                pltpu.sync_copy(srcm.at[pl.ds(lo + g * ag, ag)],
                                sidx.at[b])
                pltpu.sync_copy(dstm.at[pl.ds(lo + g * ag, ag)],
                                didx.at[b])

            def start_gathers(g, b):
                return [
                    pltpu.async_copy(table.at[sidx.at[b, j]], rows.at[b, j],
                                     gsem[b])
                    for j in range(ag)
                ]

            load_group(0, 0)
            gd = {0: start_gathers(0, 0)}
            sd = [None, None]
            for g in range(ng):
                b = g & 1
                nb = 1 - b
                if g == 0:
                    for zd in zds:
                        zd.wait()
                    plsc.subcore_barrier()
                if g + 1 < ng:
                    if sd[nb] is not None:
                        for dsc in sd[nb]:
                            dsc.wait()
                        sd[nb] = None
                    load_group(g + 1, nb)
                    gd[g + 1] = start_gathers(g + 1, nb)
                scs = []
                for j in range(ag):
                    gd[g][j].wait()
                    scs.append(pltpu.async_copy(
                        rows.at[b, j], accum.at[didx.at[b, j]], ssem[b],
                        add=True))
                sd[b] = scs
            for b in (0, 1):
                if sd[b] is not None:
                    for dsc in sd[b]:
                        dsc.wait()

        @pl.when(cid == 0)
        def _():
            edge_pipeline(sid * (ng0 * ag), ng0)

        @pl.when(cid == 1)
        def _():
            edge_pipeline(16 * ng0 * ag + sid * (ng1 * ag), ng1)

        plsc.subcore_barrier()
        wds = [
            pltpu.async_copy(
                accum.at[pl.ds(sid * RPT + k * ZR, ZR)],
                out_hbm.at[cid, pl.ds(sid * RPT + k * ZR, ZR)], zsem)
            for k in range(NZCH)
        ]
        for wd in wds:
            wd.wait()

    return agg


_agg32 = _make_agg(DH, 19, 1, 8)
_agg16 = _make_agg(DO, 9, 1, 16)


def _tc_prep(parts, x, W1):
    def body(parts_ref, x_ref, w_ref, hp_ref, dinv_ref):
        ones = jnp.ones((NW, 1), jnp.float32)
        deg = lax.dot_general(
            parts_ref[...], ones, (((0,), (0,)), ((), ())),
            preferred_element_type=jnp.float32)
        dinv = lax.rsqrt(deg[:NN] + 1.0)
        h = jnp.dot(x_ref[...], w_ref[...], preferred_element_type=jnp.float32)
        hp_ref[:NN] = h * dinv
        hp_ref[NN:] = jnp.zeros((NNP - NN, DH), jnp.float32)
        dinv_ref[...] = dinv

    return pl.pallas_call(
        body,
        out_shape=[
            jax.ShapeDtypeStruct((NNP, DH), jnp.float32),
            jax.ShapeDtypeStruct((NN, 1), jnp.float32),
        ],
    )(parts, x, W1)


def _tc_mid(acc, hp1, dinv, b1, W2):
    def body(acc_ref, hp_ref, dinv_ref, b_ref, w_ref, out_ref):
        acc = acc_ref[...]
        s = acc[0, :NN] + acc[1, :NN] + hp_ref[:NN]
        h1 = jnp.maximum(dinv_ref[...] * s + b_ref[...], 0.0)
        out_ref[:NN] = (
            jnp.dot(h1, w_ref[...], preferred_element_type=jnp.float32)
            * dinv_ref[...])
        out_ref[NN:] = jnp.zeros((NNP - NN, DO), jnp.float32)

    return pl.pallas_call(
        body,
        out_shape=jax.ShapeDtypeStruct((NNP, DO), jnp.float32),
    )(acc, hp1, dinv, b1, W2)


def _tc_final(acc, hp2, dinv, b2):
    def body(acc_ref, hp_ref, dinv_ref, b_ref, out_ref):
        acc = acc_ref[...]
        z = (dinv_ref[...] * (acc[0, :NN] + acc[1, :NN] + hp_ref[:NN])
             + b_ref[...])
        m = jnp.max(z, axis=1, keepdims=True)
        lse = jnp.log(jnp.sum(jnp.exp(z - m), axis=1, keepdims=True)) + m
        out_ref[...] = z - lse

    return pl.pallas_call(
        body,
        out_shape=jax.ShapeDtypeStruct((NN, DO), jnp.float32),
    )(acc, hp2, dinv, b2)


def kernel(x, edge_index, W1, b1, W2, b2):
    pad = jnp.full((EPAD,), NN, jnp.int32)
    srcm = jnp.concatenate([edge_index[0], pad]).reshape(EROWS, 128)
    dstm = jnp.concatenate([edge_index[1], pad]).reshape(EROWS, 128)
    parts = _deg_kernel(dstm)
    hp1, dinv = _tc_prep(parts, x, W1)
    acc1 = _agg32(hp1, srcm, dstm)
    hp2 = _tc_mid(acc1, hp1, dinv, b1.reshape(1, DH), W2)
    acc2 = _agg16(hp2, srcm, dstm)
    return _tc_final(acc2, hp2, dinv, b2.reshape(1, DO))


# speedup vs baseline: 1.0672x; 1.0077x over previous
import functools

import jax
import jax.numpy as jnp
from jax import lax
from jax.experimental import pallas as pl
from jax.experimental.pallas import tpu as pltpu
from jax.experimental.pallas import tpu_sc as plsc

NN = 10000
NE = 320000
DIN = 128
DH = 32
DO = 16

NC = 2
NS = 16
LANES = 16
NW = NC * NS
NNP = 10240
RPT = NNP // NS

EROWS = 2560
EPAD = EROWS * 128 - NE
RPW = EROWS // NW

_MESH = dict(core_axis_name="c", subcore_axis_name="s")


def _worker_id():
    return lax.axis_index("s") * NC + lax.axis_index("c")


DEG_G = 8
DEG_NG0 = 14
DEG_NG1 = 6


@functools.partial(
    pl.kernel,
    out_type=jax.ShapeDtypeStruct((NW, NNP), jnp.float32),
    mesh=plsc.VectorSubcoreMesh(**_MESH),
    scratch_types=[
        pltpu.VMEM((NNP,), jnp.float32),
        pltpu.VMEM((2, DEG_G, 128), jnp.int32),
        pltpu.SemaphoreType.DMA,
        pltpu.SemaphoreType.DMA,
    ],
    compiler_params=pltpu.CompilerParams(needs_layout_passes=False),
)
def _deg_kernel(dstm, out_hbm, hist, didx, lsem0, lsem1):
    wid = _worker_id()
    cid = lax.axis_index("c")
    sid = lax.axis_index("s")
    lsem = (lsem0, lsem1)
    zero16 = jnp.zeros((LANES,), jnp.float32)
    ones16 = jnp.ones((LANES,), jnp.float32)

    def zbody(i, c):
        hist[pl.ds(i * LANES, LANES)] = zero16
        return c

    lax.fori_loop(0, NNP // LANES, zbody, 0, unroll=8)

    def hist_pipeline(lo, ng):
        pltpu.sync_copy(dstm.at[pl.ds(lo, DEG_G)], didx.at[0])
        descs = {}
        for g in range(ng):
            b = g & 1
            if g > 0:
                descs[g].wait()
            if g + 1 < ng:
                descs[g + 1] = pltpu.async_copy(
                    dstm.at[pl.ds(lo + (g + 1) * DEG_G, DEG_G)],
                    didx.at[1 - b], lsem[1 - b])
            for j in range(DEG_G):
                for k in range(128 // LANES):
                    idx = didx[b, j, pl.ds(k * LANES, LANES)]
                    plsc.addupdate_scatter(hist, [idx], ones16)

    @pl.when(cid == 0)
    def _():
        hist_pipeline(sid * (DEG_NG0 * DEG_G), DEG_NG0)

    @pl.when(cid == 1)
    def _():
        hist_pipeline(16 * DEG_NG0 * DEG_G + sid * (DEG_NG1 * DEG_G), DEG_NG1)

    pltpu.sync_copy(hist, out_hbm.at[wid])


NZCH = 4
ZR = RPT // NZCH


def _make_agg(d, ng0, ng1, ag):
    @functools.partial(
        pl.kernel,
        out_type=jax.ShapeDtypeStruct((NC, NNP, d), jnp.float32),
        mesh=plsc.VectorSubcoreMesh(**_MESH),
        scratch_types=[
            pltpu.VMEM((2, ag, 128), jnp.int32),
            pltpu.VMEM((2, ag, 128), jnp.int32),
            pltpu.VMEM((2, ag, 128, d), jnp.float32),
            pltpu.VMEM((RPT, d), jnp.float32),
            pltpu.VMEM_SHARED((NNP, d), jnp.float32),
            [pltpu.SemaphoreType.DMA] * 2,
            [pltpu.SemaphoreType.DMA] * 2,
            pltpu.SemaphoreType.DMA,
        ],
        compiler_params=pltpu.CompilerParams(use_tc_tiling_on_sc=False),
    )
    def agg(table, srcm, dstm, out_hbm, sidx, didx, rows, bounce, accum,
            gsem, ssem, zsem):
        cid = lax.axis_index("c")
        sid = lax.axis_index("s")
        zero16 = jnp.zeros((LANES,), jnp.float32)

        def zbody(i, carry):
            for j in range(d // LANES):
                bounce[i, pl.ds(j * LANES, LANES)] = zero16
            return carry

        lax.fori_loop(0, RPT, zbody, 0, unroll=8)
        zds = [
            pltpu.async_copy(
                bounce.at[pl.ds(k * ZR, ZR)],
                accum.at[pl.ds(sid * RPT + k * ZR, ZR)], zsem)
            for k in range(NZCH)
        ]

        def edge_pipeline(lo, ng):
            def load_group(g, b):
                pltpu.sync_copy(srcm.at[pl.ds(lo + g * ag, ag)],
                                sidx.at[b])
                pltpu.sync_copy(dstm.at[pl.ds(lo + g * ag, ag)],
                                didx.at[b])

            def start_gathers(g, b):
                return [
                    pltpu.async_copy(table.at[sidx.at[b, j]], rows.at[b, j],
                                     gsem[b])
                    for j in range(ag)
                ]

            load_group(0, 0)
            gd = {0: start_gathers(0, 0)}
            sd = [None, None]
            for g in range(ng):
                b = g & 1
                nb = 1 - b
                if g == 0:
                    for zd in zds:
                        zd.wait()
                    plsc.subcore_barrier()
                if g + 1 < ng:
                    if sd[nb] is not None:
                        for dsc in sd[nb]:
                            dsc.wait()
                        sd[nb] = None
                    load_group(g + 1, nb)
                    gd[g + 1] = start_gathers(g + 1, nb)
                scs = []
                for j in range(ag):
                    gd[g][j].wait()
                    scs.append(pltpu.async_copy(
                        rows.at[b, j], accum.at[didx.at[b, j]], ssem[b],
                        add=True))
                sd[b] = scs
            for b in (0, 1):
                if sd[b] is not None:
                    for dsc in sd[b]:
                        dsc.wait()

        @pl.when(cid == 0)
        def _():
            edge_pipeline(sid * (ng0 * ag), ng0)

        @pl.when(cid == 1)
        def _():
            edge_pipeline(16 * ng0 * ag + sid * (ng1 * ag), ng1)

        plsc.subcore_barrier()
        wds = [
            pltpu.async_copy(
                accum.at[pl.ds(sid * RPT + k * ZR, ZR)],
                out_hbm.at[cid, pl.ds(sid * RPT + k * ZR, ZR)], zsem)
            for k in range(NZCH)
        ]
        for wd in wds:
            wd.wait()

    return agg


_agg32 = _make_agg(DH, 19, 1, 8)
_agg16 = _make_agg(DO, 9, 1, 16)


def _tc_prep(parts, x, W1):
    def body(parts_ref, x_ref, w_ref, hp_ref, dinv_ref):
        ones = jnp.ones((NW, 1), jnp.float32)
        deg = lax.dot_general(
            parts_ref[...], ones, (((0,), (0,)), ((), ())),
            preferred_element_type=jnp.float32)
        dinv = lax.rsqrt(deg[:NN] + 1.0)
        h = jnp.dot(x_ref[...], w_ref[...], preferred_element_type=jnp.float32)
        hp_ref[:NN] = h * dinv
        hp_ref[NN:] = jnp.zeros((NNP - NN, DH), jnp.float32)
        dinv_ref[...] = dinv

    return pl.pallas_call(
        body,
        out_shape=[
            jax.ShapeDtypeStruct((NNP, DH), jnp.float32),
            jax.ShapeDtypeStruct((NN, 1), jnp.float32),
        ],
    )(parts, x, W1)


def _tc_mid(acc, hp1, dinv, b1, W2):
    def body(acc_ref, hp_ref, dinv_ref, b_ref, w_ref, out_ref):
        acc = acc_ref[...]
        s = acc[0, :NN] + acc[1, :NN] + hp_ref[:NN]
        h1 = jnp.maximum(dinv_ref[...] * s + b_ref[...], 0.0)
        out_ref[:NN] = (
            jnp.dot(h1, w_ref[...], preferred_element_type=jnp.float32)
            * dinv_ref[...])
        out_ref[NN:] = jnp.zeros((NNP - NN, DO), jnp.float32)

    return pl.pallas_call(
        body,
        out_shape=jax.ShapeDtypeStruct((NNP, DO), jnp.float32),
    )(acc, hp1, dinv, b1, W2)


def _tc_final(acc, hp2, dinv, b2):
    def body(acc_ref, hp_ref, dinv_ref, b_ref, out_ref):
        acc = acc_ref[...]
        z = (dinv_ref[...] * (acc[0, :NN] + acc[1, :NN] + hp_ref[:NN])
             + b_ref[...])
        m = jnp.max(z, axis=1, keepdims=True)
        lse = jnp.log(jnp.sum(jnp.exp(z - m), axis=1, keepdims=True)) + m
        out_ref[...] = z - lse

    return pl.pallas_call(
        body,
        out_shape=jax.ShapeDtypeStruct((NN, DO), jnp.float32),
    )(acc, hp2, dinv, b2)


def kernel(x, edge_index, W1, b1, W2, b2):
    pad = jnp.full((EPAD,), NN, jnp.int32)
    srcm = jnp.concatenate([edge_index[0], pad]).reshape(EROWS, 128)
    dstm = jnp.concatenate([edge_index[1], pad]).reshape(EROWS, 128)
    parts = _deg_kernel(dstm)
    hp1, dinv = _tc_prep(parts, x, W1)
    acc1 = _agg32(hp1, srcm, dstm)
    hp2 = _tc_mid(acc1, hp1, dinv, b1.reshape(1, DH), W2)
    acc2 = _agg16(hp2, srcm, dstm)
    return _tc_final(acc2, hp2, dinv, b2.reshape(1, DO))
